# Initial kernel scaffold; baseline (speedup 1.0000x reference)
import jax
import jax.numpy as jnp
from jax.experimental import pallas as pl


def kernel(ref, dst):
    raise NotImplementedError("write your pallas kernel here")


# speedup vs baseline: 30.7740x; 30.7740x over previous
import jax
import jax.numpy as jnp
from jax import lax
from jax.experimental import pallas as pl
from jax.experimental.pallas import tpu as pltpu
from jax.experimental.pallas import tpu_sc as plsc

_BINS = 256
_SIGMA = 75.0
_NCH = 6
_NPIX = 224 * 224
_NW = 32
_CHUNK = _NPIX // _NW
_NVEC = _CHUNK // 16
_NUNITS = 2 * _NCH
_NTOT = _NCH * _NPIX


def _hist_body(ref_hbm, dst_hbm, out_hbm, stage, hist, acc):
    w = lax.axis_index("s") * 2 + lax.axis_index("c")
    lbase = lax.iota(jnp.int32, 16) * _BINS

    def zero_body(i, carry):
        hist[pl.ds(i * 16, 16)] = jnp.zeros((16,), jnp.float32)
        return carry

    lax.fori_loop(0, 16 * _BINS // 16, zero_body, 0)

    for u in range(_NUNITS):
        src = ref_hbm if u < _NCH else dst_hbm
        base = (u % _NCH) * _NPIX + w * _CHUNK
        pltpu.sync_copy(src.at[pl.ds(base, _CHUNK)], stage)

        def body(i, carry):
            v = stage[pl.ds(i * 16, 16)]
            x = v * 255.0
            k = x.astype(jnp.int32)
            frac = x - k.astype(jnp.float32)
            a = _SIGMA * frac
            e2 = 1.0 / (1.0 + jnp.exp(-a))
            e3 = 1.0 / (1.0 + jnp.exp(_SIGMA - a))
            plsc.addupdate_scatter(hist, [lbase + k], e2 - e3)
            bm = jnp.maximum(k - 1, 0)
            plsc.addupdate_scatter(hist, [lbase + bm], 1.0 - e2, mask=k >= 1)
            bp = jnp.minimum(k + 1, _BINS - 1)
            plsc.addupdate_scatter(hist, [lbase + bp], e3, mask=k <= _BINS - 2)
            return carry

        lax.fori_loop(0, _NVEC, body, 0)

        def red_body(j, carry):
            tot = jnp.zeros((16,), jnp.float32)
            for lane in range(16):
                sl = pl.ds(lane * _BINS + j * 16, 16)
                tot = tot + hist[sl]
                hist[sl] = jnp.zeros((16,), jnp.float32)
            acc[pl.ds(u * _BINS + j * 16, 16)] = tot
            return carry

        lax.fori_loop(0, _BINS // 16, red_body, 0)

    pltpu.sync_copy(acc, out_hbm.at[pl.ds(w * (_NUNITS * _BINS), _NUNITS * _BINS)])


def _table_body(h_ref, out_ref):
    h = jnp.sum(h_ref[...], axis=0)
    norm = jnp.maximum(jnp.sum(jnp.abs(h), axis=1, keepdims=True), 1e-12)
    hn = h / norm
    ii = lax.broadcasted_iota(jnp.int32, (_BINS, _BINS), 0)
    jj = lax.broadcasted_iota(jnp.int32, (_BINS, _BINS), 1)
    tri = jnp.where(ii <= jj, 1.0, 0.0)
    cdf = lax.dot_general(hn, tri, (((1,), (0,)), ((), ())),
                          preferred_element_type=jnp.float32,
                          precision=lax.Precision.HIGHEST)
    cr = cdf[:_NCH]
    cd = cdf[_NCH:]
    cnt = jnp.sum(jnp.where(cr[:, :, None] - cd[:, None, :] >= 0.0, 1.0, 0.0),
                  axis=1)
    out_ref[...] = jnp.clip(cnt - 1.0, 0.0, 255.0) / 255.0


def _loss_body(dst_hbm, tbl_hbm, out_hbm, stage, tbl, accv):
    w = lax.axis_index("s") * 2 + lax.axis_index("c")
    pltpu.sync_copy(tbl_hbm, tbl)
    acc = jnp.zeros((16,), jnp.float32)
    for ch in range(_NCH):
        base = ch * _NPIX + w * _CHUNK
        pltpu.sync_copy(dst_hbm.at[pl.ds(base, _CHUNK)], stage)

        def body(i, a):
            v = stage[pl.ds(i * 16, 16)]
            idx = jnp.clip((v * 255.0).astype(jnp.int32), 0, _BINS - 1)
            t = plsc.load_gather(tbl, [idx + ch * _BINS])
            return a + jnp.abs(v - t)

        acc = lax.fori_loop(0, _NVEC, body, acc)
    accv[...] = acc
    pltpu.sync_copy(accv, out_hbm.at[pl.ds(w * 16, 16)])


def kernel(ref, dst):
    rf = ref.reshape(-1)
    df = dst.reshape(-1)
    mesh = plsc.VectorSubcoreMesh(core_axis_name="c", subcore_axis_name="s")
    hists = pl.kernel(
        _hist_body,
        out_type=jax.ShapeDtypeStruct((_NW * _NUNITS * _BINS,), jnp.float32),
        mesh=mesh,
        scratch_types=[
            pltpu.VMEM((_CHUNK,), jnp.float32),
            pltpu.VMEM((16 * _BINS,), jnp.float32),
            pltpu.VMEM((_NUNITS * _BINS,), jnp.float32),
        ],
        compiler_params=pltpu.CompilerParams(needs_layout_passes=False),
    )(rf, df)
    table = pl.pallas_call(
        _table_body,
        out_shape=jax.ShapeDtypeStruct((_NCH, _BINS), jnp.float32),
    )(hists.reshape(_NW, _NUNITS, _BINS))
    parts = pl.kernel(
        _loss_body,
        out_type=jax.ShapeDtypeStruct((_NW * 16,), jnp.float32),
        mesh=mesh,
        scratch_types=[
            pltpu.VMEM((_CHUNK,), jnp.float32),
            pltpu.VMEM((_NCH * _BINS,), jnp.float32),
            pltpu.VMEM((16,), jnp.float32),
        ],
        compiler_params=pltpu.CompilerParams(needs_layout_passes=False),
    )(df, table.reshape(-1))
    return jnp.sum(parts) / float(_NTOT)


# speedup vs baseline: 33.1861x; 1.0784x over previous
import jax
import jax.numpy as jnp
from jax import lax
from jax.experimental import pallas as pl
from jax.experimental.pallas import tpu as pltpu
from jax.experimental.pallas import tpu_sc as plsc

_BINS = 256
_SIGMA = 75.0
_NCH = 6
_NPIX = 224 * 224
_NW = 32
_CHUNK = _NPIX // _NW
_NVEC = _CHUNK // 16
_NUNITS = 2 * _NCH
_NTOT = _NCH * _NPIX


_E75 = 3.7332419967990015e32
_UNROLL = 7


def _hist_body(ref_hbm, dst_hbm, out_hbm, st0, st1, hist, acc, sm0, sm1):
    w = lax.axis_index("s") * 2 + lax.axis_index("c")
    lbase = lax.iota(jnp.int32, 16) * _BINS
    bufs, sems = (st0, st1), (sm0, sm1)

    def mk_copy(u):
        src = ref_hbm if u < _NCH else dst_hbm
        base = (u % _NCH) * _NPIX + w * _CHUNK
        return pltpu.make_async_copy(
            src.at[pl.ds(base, _CHUNK)], bufs[u % 2], sems[u % 2])

    mk_copy(0).start()

    def zero_body(i, carry):
        hist[pl.ds(i * 16, 16)] = jnp.zeros((16,), jnp.float32)
        return carry

    lax.fori_loop(0, 16 * _BINS // 16, zero_body, 0)

    for u in range(_NUNITS):
        if u + 1 < _NUNITS:
            mk_copy(u + 1).start()
        mk_copy(u).wait()
        stage = bufs[u % 2]

        def body(i, carry):
            for t in range(_UNROLL):
                v = stage[pl.ds(i * (16 * _UNROLL) + t * 16, 16)]
                x = v * 255.0
                k = x.astype(jnp.int32)
                frac = x - k.astype(jnp.float32)
                et = jnp.exp(frac * (-_SIGMA))
                p = 1.0 + et
                q = 1.0 + _E75 * et
                rd = 1.0 / (p * q)
                e2 = q * rd
                e3 = p * rd
                plsc.addupdate_scatter(hist, [lbase + k], e2 - e3)
                bm = jnp.maximum(k - 1, 0)
                plsc.addupdate_scatter(hist, [lbase + bm], 1.0 - e2, mask=k >= 1)
                bp = jnp.minimum(k + 1, _BINS - 1)
                plsc.addupdate_scatter(hist, [lbase + bp], e3, mask=k <= _BINS - 2)
            return carry

        lax.fori_loop(0, _NVEC // _UNROLL, body, 0)

        def red_body(j, carry):
            tot = jnp.zeros((16,), jnp.float32)
            for lane in range(16):
                sl = pl.ds(lane * _BINS + j * 16, 16)
                tot = tot + hist[sl]
                hist[sl] = jnp.zeros((16,), jnp.float32)
            acc[pl.ds(u * _BINS + j * 16, 16)] = tot
            return carry

        lax.fori_loop(0, _BINS // 16, red_body, 0)

    pltpu.sync_copy(acc, out_hbm.at[pl.ds(w * (_NUNITS * _BINS), _NUNITS * _BINS)])


def _table_body(h_ref, out_ref):
    h = jnp.sum(h_ref[...], axis=0)
    norm = jnp.maximum(jnp.sum(jnp.abs(h), axis=1, keepdims=True), 1e-12)
    hn = h / norm
    ii = lax.broadcasted_iota(jnp.int32, (_BINS, _BINS), 0)
    jj = lax.broadcasted_iota(jnp.int32, (_BINS, _BINS), 1)
    tri = jnp.where(ii <= jj, 1.0, 0.0)
    cdf = lax.dot_general(hn, tri, (((1,), (0,)), ((), ())),
                          preferred_element_type=jnp.float32,
                          precision=lax.Precision.HIGHEST)
    cr = cdf[:_NCH]
    cd = cdf[_NCH:]
    cnt = jnp.sum(jnp.where(cr[:, :, None] - cd[:, None, :] >= 0.0, 1.0, 0.0),
                  axis=1)
    out_ref[...] = jnp.clip(cnt - 1.0, 0.0, 255.0) / 255.0


def _loss_body(dst_hbm, tbl_hbm, out_hbm, st0, st1, tbl, accv, sm0, sm1):
    w = lax.axis_index("s") * 2 + lax.axis_index("c")
    bufs, sems = (st0, st1), (sm0, sm1)

    def mk_copy(ch):
        base = ch * _NPIX + w * _CHUNK
        return pltpu.make_async_copy(
            dst_hbm.at[pl.ds(base, _CHUNK)], bufs[ch % 2], sems[ch % 2])

    mk_copy(0).start()
    pltpu.sync_copy(tbl_hbm, tbl)
    acc = jnp.zeros((16,), jnp.float32)
    for ch in range(_NCH):
        if ch + 1 < _NCH:
            mk_copy(ch + 1).start()
        mk_copy(ch).wait()
        stage = bufs[ch % 2]

        def body(i, a):
            for t in range(_UNROLL):
                v = stage[pl.ds(i * (16 * _UNROLL) + t * 16, 16)]
                idx = jnp.clip((v * 255.0).astype(jnp.int32), 0, _BINS - 1)
                tv = plsc.load_gather(tbl, [idx + ch * _BINS])
                a = a + jnp.abs(v - tv)
            return a

        acc = lax.fori_loop(0, _NVEC // _UNROLL, body, acc)
    accv[...] = acc
    pltpu.sync_copy(accv, out_hbm.at[pl.ds(w * 16, 16)])


def kernel(ref, dst):
    rf = ref.reshape(-1)
    df = dst.reshape(-1)
    mesh = plsc.VectorSubcoreMesh(core_axis_name="c", subcore_axis_name="s")
    hists = pl.kernel(
        _hist_body,
        out_type=jax.ShapeDtypeStruct((_NW * _NUNITS * _BINS,), jnp.float32),
        mesh=mesh,
        scratch_types=[
            pltpu.VMEM((_CHUNK,), jnp.float32),
            pltpu.VMEM((_CHUNK,), jnp.float32),
            pltpu.VMEM((16 * _BINS,), jnp.float32),
            pltpu.VMEM((_NUNITS * _BINS,), jnp.float32),
            pltpu.SemaphoreType.DMA,
            pltpu.SemaphoreType.DMA,
        ],
        compiler_params=pltpu.CompilerParams(needs_layout_passes=False),
    )(rf, df)
    table = pl.pallas_call(
        _table_body,
        out_shape=jax.ShapeDtypeStruct((_NCH, _BINS), jnp.float32),
    )(hists.reshape(_NW, _NUNITS, _BINS))
    parts = pl.kernel(
        _loss_body,
        out_type=jax.ShapeDtypeStruct((_NW * 16,), jnp.float32),
        mesh=mesh,
        scratch_types=[
            pltpu.VMEM((_CHUNK,), jnp.float32),
            pltpu.VMEM((_CHUNK,), jnp.float32),
            pltpu.VMEM((_NCH * _BINS,), jnp.float32),
            pltpu.VMEM((16,), jnp.float32),
            pltpu.SemaphoreType.DMA,
            pltpu.SemaphoreType.DMA,
        ],
        compiler_params=pltpu.CompilerParams(needs_layout_passes=False),
    )(df, table.reshape(-1))
    return jnp.sum(parts) / float(_NTOT)


# speedup vs baseline: 43.3933x; 1.3076x over previous
import jax
import jax.numpy as jnp
from jax import lax
from jax.experimental import pallas as pl
from jax.experimental.pallas import tpu as pltpu
from jax.experimental.pallas import tpu_sc as plsc

_BINS = 256
_SIGMA = 75.0
_NCH = 6
_NPIX = 224 * 224
_NW = 32
_CHUNK = _NPIX // _NW
_NVEC = _CHUNK // 16
_NUNITS = 2 * _NCH
_NTOT = _NCH * _NPIX


_UNROLL = 7
_Q = 4096
_SCALE = float(255 * _Q)


def _make_lut():
    import numpy as np
    qf = (np.arange(_Q, dtype=np.float64) + 0.5) / _Q
    h1 = 1.0 / (1.0 + np.exp(_SIGMA * qf))
    e3 = 1.0 / (1.0 + np.exp(_SIGMA * (1.0 - qf)))
    return np.concatenate([h1, e3]).astype(np.float32)


_LUT = _make_lut()


def _hist_body(lut_hbm, ref_hbm, dst_hbm, out_hbm, lutv, st0, st1, hist, sm0, sm1):
    w = lax.axis_index("s") * 2 + lax.axis_index("c")
    bufs, sems = (st0, st1), (sm0, sm1)

    def mk_copy(u):
        src = ref_hbm if u < _NCH else dst_hbm
        base = (u % _NCH) * _NPIX + w * _CHUNK
        return pltpu.make_async_copy(
            src.at[pl.ds(base, _CHUNK)], bufs[u % 2], sems[u % 2])

    mk_copy(0).start()
    pltpu.sync_copy(lut_hbm, lutv)

    def zero_body(i, carry):
        hist[pl.ds(i * 16, 16)] = jnp.zeros((16,), jnp.float32)
        return carry

    lax.fori_loop(0, _NUNITS * _BINS // 16, zero_body, 0)

    for u in range(_NUNITS):
        if u + 1 < _NUNITS:
            mk_copy(u + 1).start()
        mk_copy(u).wait()
        stage = bufs[u % 2]
        hu = hist.at[pl.ds(u * _BINS, _BINS)]

        def body(i, carry):
            for t in range(_UNROLL):
                v = stage[pl.ds(i * (16 * _UNROLL) + t * 16, 16)]
                yi = (v * _SCALE).astype(jnp.int32)
                k = yi >> 12
                q = yi & (_Q - 1)
                g1 = plsc.load_gather(lutv, [q])
                g3 = plsc.load_gather(lutv, [q + _Q])
                plsc.addupdate_scatter(hu, [k], 1.0 - g1 - g3)
                bm = jnp.maximum(k - 1, 0)
                plsc.addupdate_scatter(hu, [bm], g1, mask=k >= 1)
                bp = jnp.minimum(k + 1, _BINS - 1)
                plsc.addupdate_scatter(hu, [bp], g3, mask=k <= _BINS - 2)
            return carry

        lax.fori_loop(0, _NVEC // _UNROLL, body, 0)

    pltpu.sync_copy(hist, out_hbm.at[pl.ds(w * (_NUNITS * _BINS), _NUNITS * _BINS)])


def _table_body(h_ref, out_ref):
    h = jnp.sum(h_ref[...], axis=0)
    norm = jnp.maximum(jnp.sum(jnp.abs(h), axis=1, keepdims=True), 1e-12)
    hn = h / norm
    ii = lax.broadcasted_iota(jnp.int32, (_BINS, _BINS), 0)
    jj = lax.broadcasted_iota(jnp.int32, (_BINS, _BINS), 1)
    tri = jnp.where(ii <= jj, 1.0, 0.0)
    cdf = lax.dot_general(hn, tri, (((1,), (0,)), ((), ())),
                          preferred_element_type=jnp.float32,
                          precision=lax.Precision.HIGHEST)
    cr = cdf[:_NCH]
    cd = cdf[_NCH:]
    cnt = jnp.sum(jnp.where(cr[:, :, None] - cd[:, None, :] >= 0.0, 1.0, 0.0),
                  axis=1)
    out_ref[...] = jnp.clip(cnt - 1.0, 0.0, 255.0) / 255.0


def _loss_body(dst_hbm, tbl_hbm, out_hbm, st0, st1, tbl, accv, sm0, sm1):
    w = lax.axis_index("s") * 2 + lax.axis_index("c")
    bufs, sems = (st0, st1), (sm0, sm1)

    def mk_copy(ch):
        base = ch * _NPIX + w * _CHUNK
        return pltpu.make_async_copy(
            dst_hbm.at[pl.ds(base, _CHUNK)], bufs[ch % 2], sems[ch % 2])

    mk_copy(0).start()
    pltpu.sync_copy(tbl_hbm, tbl)
    acc = jnp.zeros((16,), jnp.float32)
    for ch in range(_NCH):
        if ch + 1 < _NCH:
            mk_copy(ch + 1).start()
        mk_copy(ch).wait()
        stage = bufs[ch % 2]

        def body(i, a):
            for t in range(_UNROLL):
                v = stage[pl.ds(i * (16 * _UNROLL) + t * 16, 16)]
                idx = jnp.clip((v * 255.0).astype(jnp.int32), 0, _BINS - 1)
                tv = plsc.load_gather(tbl, [idx + ch * _BINS])
                a = a + jnp.abs(v - tv)
            return a

        acc = lax.fori_loop(0, _NVEC // _UNROLL, body, acc)
    accv[...] = acc
    pltpu.sync_copy(accv, out_hbm.at[pl.ds(w * 16, 16)])


def kernel(ref, dst):
    rf = ref.reshape(-1)
    df = dst.reshape(-1)
    mesh = plsc.VectorSubcoreMesh(core_axis_name="c", subcore_axis_name="s")
    hists = pl.kernel(
        _hist_body,
        out_type=jax.ShapeDtypeStruct((_NW * _NUNITS * _BINS,), jnp.float32),
        mesh=mesh,
        scratch_types=[
            pltpu.VMEM((2 * _Q,), jnp.float32),
            pltpu.VMEM((_CHUNK,), jnp.float32),
            pltpu.VMEM((_CHUNK,), jnp.float32),
            pltpu.VMEM((_NUNITS * _BINS,), jnp.float32),
            pltpu.SemaphoreType.DMA,
            pltpu.SemaphoreType.DMA,
        ],
        compiler_params=pltpu.CompilerParams(needs_layout_passes=False),
    )(jnp.asarray(_LUT), rf, df)
    table = pl.pallas_call(
        _table_body,
        out_shape=jax.ShapeDtypeStruct((_NCH, _BINS), jnp.float32),
    )(hists.reshape(_NW, _NUNITS, _BINS))
    parts = pl.kernel(
        _loss_body,
        out_type=jax.ShapeDtypeStruct((_NW * 16,), jnp.float32),
        mesh=mesh,
        scratch_types=[
            pltpu.VMEM((_CHUNK,), jnp.float32),
            pltpu.VMEM((_CHUNK,), jnp.float32),
            pltpu.VMEM((_NCH * _BINS,), jnp.float32),
            pltpu.VMEM((16,), jnp.float32),
            pltpu.SemaphoreType.DMA,
            pltpu.SemaphoreType.DMA,
        ],
        compiler_params=pltpu.CompilerParams(needs_layout_passes=False),
    )(df, table.reshape(-1))
    return jnp.sum(parts) / float(_NTOT)
